# baseline (speedup 1.0000x reference)
import functools

import jax
import jax.numpy as jnp
from jax import lax
from jax.experimental import pallas as pl
from jax.experimental.pallas import tpu as pltpu
from jax.experimental.pallas import tpu_sc as plsc

_RADIUS = 0.1
_K = 32
_NCORES = 2
_NSUB = 16
_NWORKERS = _NCORES * _NSUB
_LANES = 16


def _worker_id():
    return lax.axis_index("s") * _NCORES + lax.axis_index("c")


def _splat_i32(x):
    return jnp.full((_LANES,), x, jnp.int32)


def _ball_body(N, S, s_per_w, xyzt, ctrt, idx_out, xv, yv, zv, cxv, cyv, czv,
               sbuf, iout):
    w = _worker_id()
    per_b = S // s_per_w
    b = w // per_b
    s0 = (w % per_b) * s_per_w

    pltpu.sync_copy(xyzt.at[b, 0], xv)
    pltpu.sync_copy(xyzt.at[b, 1], yv)
    pltpu.sync_copy(xyzt.at[b, 2], zv)
    pltpu.sync_copy(ctrt.at[b, 0, pl.ds(s0, s_per_w)], cxv)
    pltpu.sync_copy(ctrt.at[b, 1, pl.ds(s0, s_per_w)], cyv)
    pltpu.sync_copy(ctrt.at[b, 2, pl.ds(s0, s_per_w)], czv)

    iota = lax.iota(jnp.int32, _LANES)
    zeros_i = jnp.zeros((_LANES,), jnp.int32)
    r2 = jnp.float32(_RADIUS * _RADIUS)
    nchunks = N // _LANES

    def centroid_body(i, carry):
        isp = _splat_i32(i)
        cx = plsc.load_gather(cxv, [isp])
        cy = plsc.load_gather(cyv, [isp])
        cz = plsc.load_gather(czv, [isp])
        sbuf[pl.ds(0, _LANES)] = zeros_i

        def chunk(j, fv):
            base = j * _LANES
            dx = xv[pl.ds(base, _LANES)] - cx
            dy = yv[pl.ds(base, _LANES)] - cy
            dz = zv[pl.ds(base, _LANES)] - cz
            d2 = dx * dx + dy * dy + dz * dz
            m = d2 < r2
            cs = plsc.cumsum(m.astype(jnp.int32))
            pos = jnp.minimum(fv + cs - 1, 47)
            plsc.store_scatter(sbuf, [pos], iota + base, mask=m)
            return fv + plsc.all_reduce_population_count(m)

        found = plsc.parallel_loop(0, nchunks, carry=zeros_i, unroll=4)(chunk)
        first = plsc.load_gather(sbuf, [jnp.minimum(found, 0)])
        for h in range(_K // _LANES):
            cur = sbuf[pl.ds(h * _LANES, _LANES)]
            posv = iota + h * _LANES
            iout[i, pl.ds(h * _LANES, _LANES)] = jnp.where(
                posv < found, cur, first)
        return carry

    lax.fori_loop(0, s_per_w, centroid_body, 0)
    pltpu.sync_copy(iout, idx_out.at[b, pl.ds(s0, s_per_w), :])


def _gather_body(N, S, CH, xyzt, feats, ctrt, idx_in, out, idxv, row0, row1,
                 ctrv, out0, out1, lsem, ssem0, ssem1):
    w = _worker_id()
    per_b = _NWORKERS // feats.shape[0]
    b = w // per_b
    g = w % per_b
    nfeat = (CH - 3) // per_b

    pltpu.sync_copy(idx_in.at[b], idxv)
    rowbufs = [row0, row1]
    outbufs = [out0, out1]
    osem = [ssem0, ssem1]

    iota = lax.iota(jnp.int32, _LANES)
    njv = S * _K // _LANES

    def gather_into(rowv, outv, subtract=None):
        def j_body(j):
            k = j >> 6
            sb = j & 63
            st = sb >> 3
            ss0 = (sb & 7) * _LANES
            s_ids = iota + (st * 128 + ss0)
            ids = plsc.load_gather(idxv, [s_ids, _splat_i32(k)])
            vals = plsc.load_gather(rowv, [ids])
            if subtract is not None:
                vals = vals - plsc.load_gather(subtract, [s_ids])
            outv[k >> 3, st, k & 7, pl.ds(ss0, _LANES)] = vals
        plsc.parallel_loop(0, njv, unroll=8)(j_body)

    pltpu.async_copy(feats.at[b, g], row0, lsem).wait()
    for ci in range(nfeat):
        cur = ci % 2
        c = 3 + g + ci * per_b
        if ci + 1 < nfeat:
            ncopy = pltpu.make_async_copy(
                feats.at[b, c + per_b - 3], rowbufs[1 - cur], lsem)
            ncopy.start()
        if ci >= 2:
            pltpu.make_async_copy(
                outbufs[cur], out.at[b, c], osem[cur]).wait()
        gather_into(rowbufs[cur], outbufs[cur])
        pltpu.make_async_copy(outbufs[cur], out.at[b, c], osem[cur]).start()
        if ci + 1 < nfeat:
            ncopy.wait()
    pltpu.make_async_copy(out0, out.at[b, 3 + g], ssem0).wait()
    pltpu.make_async_copy(out1, out.at[b, 3 + g], ssem1).wait()

    @pl.when(g < 3)
    def _():
        gsafe = jnp.minimum(g, 2)
        pltpu.sync_copy(xyzt.at[b, gsafe], row0)
        pltpu.sync_copy(ctrt.at[b, gsafe], ctrv)
        gather_into(row0, out0, subtract=ctrv)
        pltpu.sync_copy(out0, out.at[b, gsafe])


@jax.jit
def kernel(xyz, new_xyz, features):
    B, N, _ = xyz.shape
    S = new_xyz.shape[1]
    C = features.shape[1]
    CH = C + 3
    s_per_w = S // (_NWORKERS // B)

    xyzt = jnp.transpose(xyz, (0, 2, 1))
    ctrt = jnp.transpose(new_xyz, (0, 2, 1))

    mesh = plsc.VectorSubcoreMesh(core_axis_name="c", subcore_axis_name="s")
    cparams = pltpu.CompilerParams(
        use_tc_tiling_on_sc=False, needs_layout_passes=False)

    ball = pl.kernel(
        functools.partial(_ball_body, N, S, s_per_w),
        out_type=jax.ShapeDtypeStruct((B, S, _K), jnp.int32),
        mesh=mesh,
        scratch_types=[
            pltpu.VMEM((N,), jnp.float32),
            pltpu.VMEM((N,), jnp.float32),
            pltpu.VMEM((N,), jnp.float32),
            pltpu.VMEM((s_per_w,), jnp.float32),
            pltpu.VMEM((s_per_w,), jnp.float32),
            pltpu.VMEM((s_per_w,), jnp.float32),
            pltpu.VMEM((48,), jnp.int32),
            pltpu.VMEM((s_per_w, _K), jnp.int32),
        ],
        compiler_params=cparams,
    )
    idx = ball(xyzt, ctrt)

    tile = (_K // 8, S // 128, 8, 128)
    gather = pl.kernel(
        functools.partial(_gather_body, N, S, CH),
        out_type=jax.ShapeDtypeStruct((B, CH) + tile, jnp.float32),
        mesh=mesh,
        scratch_types=[
            pltpu.VMEM((S, _K), jnp.int32),
            pltpu.VMEM((N,), jnp.float32),
            pltpu.VMEM((N,), jnp.float32),
            pltpu.VMEM((S,), jnp.float32),
            pltpu.VMEM(tile, jnp.float32),
            pltpu.VMEM(tile, jnp.float32),
            pltpu.SemaphoreType.DMA,
            pltpu.SemaphoreType.DMA,
            pltpu.SemaphoreType.DMA,
        ],
        compiler_params=cparams,
    )
    out6 = gather(xyzt, features, ctrt, idx)
    out = out6.transpose(0, 1, 2, 4, 3, 5).reshape(B, CH, _K, S)
    return out.transpose(0, 1, 3, 2)


# speedup vs baseline: 1.8366x; 1.8366x over previous
import functools

import jax
import jax.numpy as jnp
from jax import lax
from jax.experimental import pallas as pl
from jax.experimental.pallas import tpu as pltpu
from jax.experimental.pallas import tpu_sc as plsc

_RADIUS = 0.1
_K = 32
_NCORES = 2
_NSUB = 16
_NWORKERS = _NCORES * _NSUB
_LANES = 16


def _worker_id():
    return lax.axis_index("s") * _NCORES + lax.axis_index("c")


def _splat_i32(x):
    return jnp.full((_LANES,), x, jnp.int32)


def _ball_body(N, S, s_per_w, xyzt, ctrt, idx_out, xv, yv, zv, cxv, cyv, czv,
               sbuf, iout):
    w = _worker_id()
    per_b = S // s_per_w
    b = w // per_b
    s0 = (w % per_b) * s_per_w

    pltpu.sync_copy(xyzt.at[b, 0], xv)
    pltpu.sync_copy(xyzt.at[b, 1], yv)
    pltpu.sync_copy(xyzt.at[b, 2], zv)
    pltpu.sync_copy(ctrt.at[b, 0, pl.ds(s0, s_per_w)], cxv)
    pltpu.sync_copy(ctrt.at[b, 1, pl.ds(s0, s_per_w)], cyv)
    pltpu.sync_copy(ctrt.at[b, 2, pl.ds(s0, s_per_w)], czv)

    iota = lax.iota(jnp.int32, _LANES)
    zeros_i = jnp.zeros((_LANES,), jnp.int32)
    r2 = jnp.float32(_RADIUS * _RADIUS)
    nchunks = N // _LANES

    def centroid_body(i, carry):
        isp = _splat_i32(i)
        cx = plsc.load_gather(cxv, [isp])
        cy = plsc.load_gather(cyv, [isp])
        cz = plsc.load_gather(czv, [isp])
        sbuf[pl.ds(0, _LANES)] = zeros_i

        def chunk(j, fv):
            base = j * _LANES
            dx = xv[pl.ds(base, _LANES)] - cx
            dy = yv[pl.ds(base, _LANES)] - cy
            dz = zv[pl.ds(base, _LANES)] - cz
            d2 = dx * dx + dy * dy + dz * dz
            m = d2 < r2
            cs = plsc.cumsum(m.astype(jnp.int32))
            pos = jnp.minimum(fv + cs - 1, 47)
            plsc.store_scatter(sbuf, [pos], iota + base, mask=m)
            return fv + plsc.all_reduce_population_count(m)

        found = plsc.parallel_loop(0, nchunks, carry=zeros_i, unroll=4)(chunk)
        first = plsc.load_gather(sbuf, [jnp.minimum(found, 0)])
        for h in range(_K // _LANES):
            cur = sbuf[pl.ds(h * _LANES, _LANES)]
            posv = iota + h * _LANES
            res = jnp.where(posv < found, cur, first)
            plsc.store_scatter(iout, [posv, _splat_i32(i)], res)
        return carry

    lax.fori_loop(0, s_per_w, centroid_body, 0)
    pltpu.sync_copy(iout, idx_out.at[b, :, pl.ds(s0, s_per_w)])


def _gather_body(N, S, CH, xyzt, feats, ctrt, idx_in, out, idxv, row0, row1,
                 ctrv, out0, out1, lsem, ssem0, ssem1):
    w = _worker_id()
    per_b = _NWORKERS // feats.shape[0]
    b = w // per_b
    g = w % per_b
    nfeat = (CH - 3) // per_b

    pltpu.sync_copy(idx_in.at[b], idxv)
    rowbufs = [row0, row1]
    outbufs = [out0, out1]
    osem = [ssem0, ssem1]

    iota = lax.iota(jnp.int32, _LANES)
    njv = S * _K // _LANES

    def gather_into(rowv, outv, subtract=None):
        def j_body(j):
            k = j >> 6
            sb = j & 63
            st = sb >> 3
            ss0 = (sb & 7) * _LANES
            ids = idxv[k, pl.ds(sb * _LANES, _LANES)]
            vals = plsc.load_gather(rowv, [ids])
            if subtract is not None:
                vals = vals - subtract[pl.ds(sb * _LANES, _LANES)]
            outv[k >> 3, st, k & 7, pl.ds(ss0, _LANES)] = vals
        plsc.parallel_loop(0, njv, unroll=8)(j_body)

    pltpu.async_copy(feats.at[b, g], row0, lsem).wait()
    for ci in range(nfeat):
        cur = ci % 2
        c = 3 + g + ci * per_b
        if ci + 1 < nfeat:
            ncopy = pltpu.make_async_copy(
                feats.at[b, c + per_b - 3], rowbufs[1 - cur], lsem)
            ncopy.start()
        if ci >= 2:
            pltpu.make_async_copy(
                outbufs[cur], out.at[b, c], osem[cur]).wait()
        gather_into(rowbufs[cur], outbufs[cur])
        pltpu.make_async_copy(outbufs[cur], out.at[b, c], osem[cur]).start()
        if ci + 1 < nfeat:
            ncopy.wait()
    pltpu.make_async_copy(out0, out.at[b, 3 + g], ssem0).wait()
    pltpu.make_async_copy(out1, out.at[b, 3 + g], ssem1).wait()

    @pl.when(g < 3)
    def _():
        gsafe = jnp.minimum(g, 2)
        pltpu.sync_copy(xyzt.at[b, gsafe], row0)
        pltpu.sync_copy(ctrt.at[b, gsafe], ctrv)
        gather_into(row0, out0, subtract=ctrv)
        pltpu.sync_copy(out0, out.at[b, gsafe])


@jax.jit
def kernel(xyz, new_xyz, features):
    B, N, _ = xyz.shape
    S = new_xyz.shape[1]
    C = features.shape[1]
    CH = C + 3
    s_per_w = S // (_NWORKERS // B)

    xyzt = jnp.transpose(xyz, (0, 2, 1))
    ctrt = jnp.transpose(new_xyz, (0, 2, 1))

    mesh = plsc.VectorSubcoreMesh(core_axis_name="c", subcore_axis_name="s")
    cparams = pltpu.CompilerParams(
        use_tc_tiling_on_sc=False, needs_layout_passes=False)

    ball = pl.kernel(
        functools.partial(_ball_body, N, S, s_per_w),
        out_type=jax.ShapeDtypeStruct((B, _K, S), jnp.int32),
        mesh=mesh,
        scratch_types=[
            pltpu.VMEM((N,), jnp.float32),
            pltpu.VMEM((N,), jnp.float32),
            pltpu.VMEM((N,), jnp.float32),
            pltpu.VMEM((s_per_w,), jnp.float32),
            pltpu.VMEM((s_per_w,), jnp.float32),
            pltpu.VMEM((s_per_w,), jnp.float32),
            pltpu.VMEM((48,), jnp.int32),
            pltpu.VMEM((_K, s_per_w), jnp.int32),
        ],
        compiler_params=cparams,
    )
    idx = ball(xyzt, ctrt)

    tile = (_K // 8, S // 128, 8, 128)
    gather = pl.kernel(
        functools.partial(_gather_body, N, S, CH),
        out_type=jax.ShapeDtypeStruct((B, CH) + tile, jnp.float32),
        mesh=mesh,
        scratch_types=[
            pltpu.VMEM((_K, S), jnp.int32),
            pltpu.VMEM((N,), jnp.float32),
            pltpu.VMEM((N,), jnp.float32),
            pltpu.VMEM((S,), jnp.float32),
            pltpu.VMEM(tile, jnp.float32),
            pltpu.VMEM(tile, jnp.float32),
            pltpu.SemaphoreType.DMA,
            pltpu.SemaphoreType.DMA,
            pltpu.SemaphoreType.DMA,
        ],
        compiler_params=cparams,
    )
    out6 = gather(xyzt, features, ctrt, idx)
    out = out6.transpose(0, 1, 2, 4, 3, 5).reshape(B, CH, _K, S)
    return out.transpose(0, 1, 3, 2)


# speedup vs baseline: 2.2577x; 1.2293x over previous
import functools

import jax
import jax.numpy as jnp
from jax import lax
from jax.experimental import pallas as pl
from jax.experimental.pallas import tpu as pltpu
from jax.experimental.pallas import tpu_sc as plsc

_RADIUS = 0.1
_K = 32
_NCORES = 2
_NSUB = 16
_NWORKERS = _NCORES * _NSUB
_LANES = 16


def _worker_id():
    return lax.axis_index("s") * _NCORES + lax.axis_index("c")


def _splat_i32(x):
    return jnp.full((_LANES,), x, jnp.int32)


def _ball_body(N, S, s_per_w, xyzt, ctrt, idx_out, xv, yv, zv, cxv, cyv, czv,
               sbuf, iout):
    w = _worker_id()
    per_b = S // s_per_w
    b = w // per_b
    s0 = (w % per_b) * s_per_w

    pltpu.sync_copy(xyzt.at[b, 0], xv)
    pltpu.sync_copy(xyzt.at[b, 1], yv)
    pltpu.sync_copy(xyzt.at[b, 2], zv)
    pltpu.sync_copy(ctrt.at[b, 0, pl.ds(s0, s_per_w)], cxv)
    pltpu.sync_copy(ctrt.at[b, 1, pl.ds(s0, s_per_w)], cyv)
    pltpu.sync_copy(ctrt.at[b, 2, pl.ds(s0, s_per_w)], czv)

    iota = lax.iota(jnp.int32, _LANES)
    zeros_i = jnp.zeros((_LANES,), jnp.int32)
    ones_i = jnp.ones((_LANES,), jnp.int32)
    r2 = jnp.float32(_RADIUS * _RADIUS)
    nchunks = N // _LANES

    def centroid_body(i, carry):
        isp = _splat_i32(i)
        cx = plsc.load_gather(cxv, [isp])
        cy = plsc.load_gather(cyv, [isp])
        cz = plsc.load_gather(czv, [isp])
        sbuf[pl.ds(0, _LANES)] = zeros_i

        def chunk(j, fv):
            base = j * _LANES
            dx = xv[pl.ds(base, _LANES)] - cx
            dy = yv[pl.ds(base, _LANES)] - cy
            dz = zv[pl.ds(base, _LANES)] - cz
            d2 = dx * dx + dy * dy + dz * dz
            m = d2 < r2
            cs = plsc.cumsum(ones_i, mask=m)
            plsc.store_scatter(sbuf, [fv + cs], iota + base, mask=m)
            return fv + plsc.all_reduce_population_count(m)

        fv = plsc.parallel_loop(
            0, nchunks, carry=zeros_i - 1, unroll=4)(chunk)
        first = plsc.load_gather(
            sbuf, [jnp.minimum(jnp.maximum(fv, 0), 0)])
        for h in range(_K // _LANES):
            cur = sbuf[pl.ds(h * _LANES, _LANES)]
            posv = iota + h * _LANES
            res = jnp.where(posv <= fv, cur, first)
            plsc.store_scatter(iout, [posv, _splat_i32(i)], res)
        return carry

    lax.fori_loop(0, s_per_w, centroid_body, 0)
    pltpu.sync_copy(iout, idx_out.at[b, :, pl.ds(s0, s_per_w)])


def _gather_body(N, S, CH, xyzt, feats, ctrt, idx_in, out, idxv, row0, row1,
                 ctrv, out0, out1, lsem, ssem0, ssem1):
    w = _worker_id()
    per_b = _NWORKERS // feats.shape[0]
    b = w // per_b
    g = w % per_b
    nfeat = (CH - 3) // per_b

    pltpu.sync_copy(idx_in.at[b], idxv)
    rowbufs = [row0, row1]
    outbufs = [out0, out1]
    osem = [ssem0, ssem1]

    iota = lax.iota(jnp.int32, _LANES)
    njv = S * _K // _LANES

    def gather_into(rowv, outv, subtract=None):
        def j_body(j):
            k = j >> 6
            sb = j & 63
            st = sb >> 3
            ss0 = (sb & 7) * _LANES
            ids = idxv[k, pl.ds(sb * _LANES, _LANES)]
            vals = plsc.load_gather(rowv, [ids])
            if subtract is not None:
                vals = vals - subtract[pl.ds(sb * _LANES, _LANES)]
            outv[k >> 3, st, k & 7, pl.ds(ss0, _LANES)] = vals
        plsc.parallel_loop(0, njv, unroll=8)(j_body)

    pltpu.async_copy(feats.at[b, g], row0, lsem).wait()
    for ci in range(nfeat):
        cur = ci % 2
        c = 3 + g + ci * per_b
        if ci + 1 < nfeat:
            ncopy = pltpu.make_async_copy(
                feats.at[b, c + per_b - 3], rowbufs[1 - cur], lsem)
            ncopy.start()
        if ci >= 2:
            pltpu.make_async_copy(
                outbufs[cur], out.at[b, c], osem[cur]).wait()
        gather_into(rowbufs[cur], outbufs[cur])
        pltpu.make_async_copy(outbufs[cur], out.at[b, c], osem[cur]).start()
        if ci + 1 < nfeat:
            ncopy.wait()
    pltpu.make_async_copy(out0, out.at[b, 3 + g], ssem0).wait()
    pltpu.make_async_copy(out1, out.at[b, 3 + g], ssem1).wait()

    @pl.when(g < 3)
    def _():
        gsafe = jnp.minimum(g, 2)
        pltpu.sync_copy(xyzt.at[b, gsafe], row0)
        pltpu.sync_copy(ctrt.at[b, gsafe], ctrv)
        gather_into(row0, out0, subtract=ctrv)
        pltpu.sync_copy(out0, out.at[b, gsafe])


@jax.jit
def kernel(xyz, new_xyz, features):
    B, N, _ = xyz.shape
    S = new_xyz.shape[1]
    C = features.shape[1]
    CH = C + 3
    s_per_w = S // (_NWORKERS // B)

    xyzt = jnp.transpose(xyz, (0, 2, 1))
    ctrt = jnp.transpose(new_xyz, (0, 2, 1))

    mesh = plsc.VectorSubcoreMesh(core_axis_name="c", subcore_axis_name="s")
    cparams = pltpu.CompilerParams(
        use_tc_tiling_on_sc=False, needs_layout_passes=False)

    ball = pl.kernel(
        functools.partial(_ball_body, N, S, s_per_w),
        out_type=jax.ShapeDtypeStruct((B, _K, S), jnp.int32),
        mesh=mesh,
        scratch_types=[
            pltpu.VMEM((N,), jnp.float32),
            pltpu.VMEM((N,), jnp.float32),
            pltpu.VMEM((N,), jnp.float32),
            pltpu.VMEM((s_per_w,), jnp.float32),
            pltpu.VMEM((s_per_w,), jnp.float32),
            pltpu.VMEM((s_per_w,), jnp.float32),
            pltpu.VMEM((N,), jnp.int32),
            pltpu.VMEM((_K, s_per_w), jnp.int32),
        ],
        compiler_params=cparams,
    )
    idx = ball(xyzt, ctrt)

    tile = (_K // 8, S // 128, 8, 128)
    gather = pl.kernel(
        functools.partial(_gather_body, N, S, CH),
        out_type=jax.ShapeDtypeStruct((B, CH) + tile, jnp.float32),
        mesh=mesh,
        scratch_types=[
            pltpu.VMEM((_K, S), jnp.int32),
            pltpu.VMEM((N,), jnp.float32),
            pltpu.VMEM((N,), jnp.float32),
            pltpu.VMEM((S,), jnp.float32),
            pltpu.VMEM(tile, jnp.float32),
            pltpu.VMEM(tile, jnp.float32),
            pltpu.SemaphoreType.DMA,
            pltpu.SemaphoreType.DMA,
            pltpu.SemaphoreType.DMA,
        ],
        compiler_params=cparams,
    )
    out6 = gather(xyzt, features, ctrt, idx)
    out = out6.transpose(0, 1, 2, 4, 3, 5).reshape(B, CH, _K, S)
    return out.transpose(0, 1, 3, 2)


# speedup vs baseline: 2.2625x; 1.0021x over previous
import functools

import jax
import jax.numpy as jnp
from jax import lax
from jax.experimental import pallas as pl
from jax.experimental.pallas import tpu as pltpu
from jax.experimental.pallas import tpu_sc as plsc

_RADIUS = 0.1
_K = 32
_NCORES = 2
_NSUB = 16
_NWORKERS = _NCORES * _NSUB
_LANES = 16


def _worker_id():
    return lax.axis_index("s") * _NCORES + lax.axis_index("c")


def _splat_i32(x):
    return jnp.full((_LANES,), x, jnp.int32)


def _ball_body(N, S, s_per_w, xyzt, ctrt, idx_out, xv, yv, zv, cxv, cyv, czv,
               sbuf, iout):
    w = _worker_id()
    per_b = S // s_per_w
    b = w // per_b
    s0 = (w % per_b) * s_per_w

    pltpu.sync_copy(xyzt.at[b, 0], xv)
    pltpu.sync_copy(xyzt.at[b, 1], yv)
    pltpu.sync_copy(xyzt.at[b, 2], zv)
    pltpu.sync_copy(ctrt.at[b, 0, pl.ds(s0, s_per_w)], cxv)
    pltpu.sync_copy(ctrt.at[b, 1, pl.ds(s0, s_per_w)], cyv)
    pltpu.sync_copy(ctrt.at[b, 2, pl.ds(s0, s_per_w)], czv)

    iota = lax.iota(jnp.int32, _LANES)
    zeros_i = jnp.zeros((_LANES,), jnp.int32)
    ones_i = jnp.ones((_LANES,), jnp.int32)
    r2 = jnp.float32(_RADIUS * _RADIUS)
    nchunks = N // _LANES

    def centroid_body(i, carry):
        isp = _splat_i32(i)
        cx = plsc.load_gather(cxv, [isp])
        cy = plsc.load_gather(cyv, [isp])
        cz = plsc.load_gather(czv, [isp])
        sbuf[pl.ds(0, _LANES)] = zeros_i

        def chunk(j, fv):
            base = j * _LANES
            dx = xv[pl.ds(base, _LANES)] - cx
            dy = yv[pl.ds(base, _LANES)] - cy
            dz = zv[pl.ds(base, _LANES)] - cz
            d2 = dx * dx + dy * dy + dz * dz
            m = d2 < r2
            cs = plsc.cumsum(ones_i, mask=m)
            plsc.store_scatter(sbuf, [fv + cs], iota + base, mask=m)
            return fv + plsc.all_reduce_population_count(m)

        fv = plsc.parallel_loop(
            0, nchunks, carry=zeros_i - 1, unroll=4)(chunk)
        first = plsc.load_gather(sbuf, [jnp.minimum(fv + 1, 0)])
        for h in range(_K // _LANES):
            cur = sbuf[pl.ds(h * _LANES, _LANES)]
            posv = iota + h * _LANES
            res = jnp.where(posv <= fv, cur, first)
            plsc.store_scatter(iout, [posv, _splat_i32(i)], res)
        return carry

    lax.fori_loop(0, s_per_w, centroid_body, 0)
    pltpu.sync_copy(iout, idx_out.at[b, :, pl.ds(s0, s_per_w)])


def _gather_body(N, S, CH, xyzt, feats, ctrt, idx_in, out, idxv, row0, row1,
                 ctrv, out0, out1, lsem, ssem0, ssem1):
    w = _worker_id()
    per_b = _NWORKERS // feats.shape[0]
    b = w // per_b
    g = w % per_b
    nfeat = (CH - 3) // per_b

    pltpu.sync_copy(idx_in.at[b], idxv)
    rowbufs = [row0, row1]
    outbufs = [out0, out1]
    osem = [ssem0, ssem1]

    iota = lax.iota(jnp.int32, _LANES)
    njv = S * _K // _LANES

    def gather_into(rowv, outv, subtract=None):
        def j_body(j):
            k = j >> 6
            sb = j & 63
            st = sb >> 3
            ss0 = (sb & 7) * _LANES
            ids = idxv[k, pl.ds(sb * _LANES, _LANES)]
            vals = plsc.load_gather(rowv, [ids])
            if subtract is not None:
                vals = vals - subtract[pl.ds(sb * _LANES, _LANES)]
            outv[k >> 3, st, k & 7, pl.ds(ss0, _LANES)] = vals
        plsc.parallel_loop(0, njv, unroll=8)(j_body)

    pltpu.async_copy(feats.at[b, g], row0, lsem).wait()
    for ci in range(nfeat):
        cur = ci % 2
        c = 3 + g + ci * per_b
        if ci + 1 < nfeat:
            ncopy = pltpu.make_async_copy(
                feats.at[b, c + per_b - 3], rowbufs[1 - cur], lsem)
            ncopy.start()
        if ci >= 2:
            pltpu.make_async_copy(
                outbufs[cur], out.at[b, c], osem[cur]).wait()
        gather_into(rowbufs[cur], outbufs[cur])
        pltpu.make_async_copy(outbufs[cur], out.at[b, c], osem[cur]).start()
        if ci + 1 < nfeat:
            ncopy.wait()
    pltpu.make_async_copy(out0, out.at[b, 3 + g], ssem0).wait()
    pltpu.make_async_copy(out1, out.at[b, 3 + g], ssem1).wait()

    @pl.when(g < 3)
    def _():
        gsafe = jnp.minimum(g, 2)
        pltpu.sync_copy(xyzt.at[b, gsafe], row0)
        pltpu.sync_copy(ctrt.at[b, gsafe], ctrv)
        gather_into(row0, out0, subtract=ctrv)
        pltpu.sync_copy(out0, out.at[b, gsafe])


@jax.jit
def kernel(xyz, new_xyz, features):
    B, N, _ = xyz.shape
    S = new_xyz.shape[1]
    C = features.shape[1]
    CH = C + 3
    s_per_w = S // (_NWORKERS // B)

    xyzt = jnp.transpose(xyz, (0, 2, 1))
    ctrt = jnp.transpose(new_xyz, (0, 2, 1))

    mesh = plsc.VectorSubcoreMesh(core_axis_name="c", subcore_axis_name="s")
    cparams = pltpu.CompilerParams(
        use_tc_tiling_on_sc=False, needs_layout_passes=False)

    ball = pl.kernel(
        functools.partial(_ball_body, N, S, s_per_w),
        out_type=jax.ShapeDtypeStruct((B, _K, S), jnp.int32),
        mesh=mesh,
        scratch_types=[
            pltpu.VMEM((N,), jnp.float32),
            pltpu.VMEM((N,), jnp.float32),
            pltpu.VMEM((N,), jnp.float32),
            pltpu.VMEM((s_per_w,), jnp.float32),
            pltpu.VMEM((s_per_w,), jnp.float32),
            pltpu.VMEM((s_per_w,), jnp.float32),
            pltpu.VMEM((N,), jnp.int32),
            pltpu.VMEM((_K, s_per_w), jnp.int32),
        ],
        compiler_params=cparams,
    )
    idx = ball(xyzt, ctrt)

    tile = (_K // 8, S // 128, 8, 128)
    gather = pl.kernel(
        functools.partial(_gather_body, N, S, CH),
        out_type=jax.ShapeDtypeStruct((B, CH) + tile, jnp.float32),
        mesh=mesh,
        scratch_types=[
            pltpu.VMEM((_K, S), jnp.int32),
            pltpu.VMEM((N,), jnp.float32),
            pltpu.VMEM((N,), jnp.float32),
            pltpu.VMEM((S,), jnp.float32),
            pltpu.VMEM(tile, jnp.float32),
            pltpu.VMEM(tile, jnp.float32),
            pltpu.SemaphoreType.DMA,
            pltpu.SemaphoreType.DMA,
            pltpu.SemaphoreType.DMA,
        ],
        compiler_params=cparams,
    )
    out6 = gather(xyzt, features, ctrt, idx)
    out = out6.transpose(0, 1, 2, 4, 3, 5).reshape(B, CH, _K, S)
    return out.transpose(0, 1, 3, 2)
